# Initial kernel scaffold; baseline (speedup 1.0000x reference)
import functools

import jax
import jax.numpy as jnp
from jax import lax
from jax.experimental import pallas as pl
from jax.experimental.pallas import tpu as pltpu
from jax.experimental.pallas import tpu_sc as plsc

N = 10000
E = 320000
D = 128
R = 8
NB = 30

NPAD = 10240
NC = 2
NS = 16
NW = NC * NS
EW = E // NW
C = 80
NCHUNK = EW // C
ROWS_PER_TILE = NPAD // NS


def _weights_body(coef_ref, basis_ref, root1_ref, out_ref):
    r = pl.program_id(0)
    acc = jnp.zeros((D, D), jnp.float32)
    for b in range(NB):
        acc = acc + coef_ref[r, b] * basis_ref[b]
    out_ref[0] = jnp.where(r == R, root1_ref[...], acc)


def _compute_w_all(coef_ext, basis, root1):
    return pl.pallas_call(
        _weights_body,
        grid=(R + 1,),
        in_specs=[
            pl.BlockSpec(memory_space=pltpu.SMEM),
            pl.BlockSpec((NB, D, D), lambda r: (0, 0, 0)),
            pl.BlockSpec((D, D), lambda r: (0, 0)),
        ],
        out_specs=pl.BlockSpec((1, D, D), lambda r: (r, 0, 0)),
        out_shape=jax.ShapeDtypeStruct((R + 1, D, D), jnp.float32),
    )(coef_ext, basis, root1)


def _matmul_body(x_ref, w_ref, out_ref):
    out_ref[0] = jnp.dot(x_ref[...], w_ref[0], preferred_element_type=jnp.float32)


def _compute_h_all(x_pad, w_all, bn=512):
    nblk = NPAD // bn
    return pl.pallas_call(
        _matmul_body,
        grid=(R + 1, nblk),
        in_specs=[
            pl.BlockSpec((bn, D), lambda r, n: (n, 0)),
            pl.BlockSpec((1, D, D), lambda r, n: (r, 0, 0)),
        ],
        out_specs=pl.BlockSpec((1, bn, D), lambda r, n: (r, n, 0)),
        out_shape=jax.ShapeDtypeStruct((R + 1, NPAD, D), jnp.float32),
    )(x_pad, w_all)


def _h1_body(p0_ref, p1_ref, hr_ref, bias_ref, out_ref):
    out_ref[...] = p0_ref[...] + p1_ref[...] + hr_ref[...] + bias_ref[...]


def _compute_h1(p0, p1, hroot, bias1, bn=512):
    nblk = NPAD // bn
    blk = pl.BlockSpec((bn, D), lambda n: (n, 0))
    return pl.pallas_call(
        _h1_body,
        grid=(nblk,),
        in_specs=[blk, blk, blk, pl.BlockSpec((1, D), lambda n: (0, 0))],
        out_specs=blk,
        out_shape=jax.ShapeDtypeStruct((NPAD, D), jnp.float32),
    )(p0, p1, hroot, bias1)


def _out_body(q0_ref, q1_ref, h1_ref, wn_ref, wr_ref, bias_ref, out_ref):
    agg = q0_ref[...] + q1_ref[...]
    out_ref[...] = (
        jnp.dot(agg, wn_ref[...], preferred_element_type=jnp.float32)
        + jnp.dot(h1_ref[...], wr_ref[...], preferred_element_type=jnp.float32)
        + bias_ref[...]
    )


def _compute_out(q0, q1, h1, wn, wr, bias2, bn=512):
    nblk = NPAD // bn
    blk = pl.BlockSpec((bn, D), lambda n: (n, 0))
    wblk = pl.BlockSpec((D, D), lambda n: (0, 0))
    return pl.pallas_call(
        _out_body,
        grid=(nblk,),
        in_specs=[blk, blk, blk, wblk, wblk, pl.BlockSpec((1, D), lambda n: (0, 0))],
        out_specs=blk,
        out_shape=jax.ShapeDtypeStruct((NPAD, D), jnp.float32),
    )(q0, q1, h1, wn, wr, bias2)


def _scatter_body(scale, table_hbm, idx_hbm, dst_hbm, norm_hbm, out_hbm,
                  idx_v, dst_v, norm_v, rows_v, zbuf_v, acc_shared, sem):
    c = lax.axis_index("c")
    s = lax.axis_index("s")
    wid = c * NS + s

    zero16 = jnp.zeros((16,), jnp.float32)

    def zrow(i, _):
        for j in range(D // 16):
            zbuf_v[i, pl.ds(j * 16, 16)] = zero16
        return 0

    lax.fori_loop(0, C, zrow, 0)
    for k in range(ROWS_PER_TILE // C):
        pltpu.sync_copy(zbuf_v, acc_shared.at[pl.ds(s * ROWS_PER_TILE + k * C, C), :])
    plsc.subcore_barrier()

    def chunk(g, _):
        base = wid * EW + g * C
        pltpu.sync_copy(idx_hbm.at[pl.ds(base, C)], idx_v)
        pltpu.sync_copy(dst_hbm.at[pl.ds(base, C)], dst_v)
        pltpu.async_copy(table_hbm.at[idx_v], rows_v, sem).wait()
        if scale:
            pltpu.sync_copy(norm_hbm.at[pl.ds(base, C)], norm_v)

            def edge(i, _):
                nv = plsc.load_gather(norm_v, [jnp.full((16,), i, jnp.int32)])
                for j in range(D // 16):
                    sl = pl.ds(j * 16, 16)
                    rows_v[i, sl] = rows_v[i, sl] * nv
                return 0

            lax.fori_loop(0, C, edge, 0)
        pltpu.sync_copy(rows_v, acc_shared.at[dst_v], add=True)
        return 0

    lax.fori_loop(0, NCHUNK, chunk, 0)
    plsc.subcore_barrier()

    pltpu.sync_copy(
        acc_shared.at[pl.ds(s * ROWS_PER_TILE, ROWS_PER_TILE), :],
        out_hbm.at[c, pl.ds(s * ROWS_PER_TILE, ROWS_PER_TILE), :],
    )


def _sc_scatter(table, idx, dst, norm, scale):
    mesh = plsc.VectorSubcoreMesh(core_axis_name="c", subcore_axis_name="s")
    kern = pl.kernel(
        functools.partial(_scatter_body, scale),
        out_type=jax.ShapeDtypeStruct((NC, NPAD, D), jnp.float32),
        mesh=mesh,
        scratch_types=[
            pltpu.VMEM((C,), jnp.int32),
            pltpu.VMEM((C,), jnp.int32),
            pltpu.VMEM((C,), jnp.float32),
            pltpu.VMEM((C, D), jnp.float32),
            pltpu.VMEM((C, D), jnp.float32),
            pltpu.VMEM_SHARED((NPAD, D), jnp.float32),
            pltpu.SemaphoreType.DMA,
        ],
    )
    return kern(table, idx, dst, norm)


def kernel(node_features, edge_index, edge_norm, edge_type, basis, coef,
           root1, bias1, w_neigh2, w_root2, bias2):
    src = edge_index[0].astype(jnp.int32)
    dst = edge_index[1].astype(jnp.int32)
    et = edge_type.astype(jnp.int32)
    idx1 = et * NPAD + src

    x_pad = jnp.pad(node_features, ((0, NPAD - N), (0, 0)))
    coef_ext = jnp.concatenate([coef, jnp.zeros((1, NB), coef.dtype)], axis=0)

    w_all = _compute_w_all(coef_ext, basis, root1)
    h_all = _compute_h_all(x_pad, w_all)
    table1 = h_all.reshape((R + 1) * NPAD, D)
    hroot = h_all[R]

    parts1 = _sc_scatter(table1, idx1, dst, edge_norm, scale=True)
    h1_pad = _compute_h1(parts1[0], parts1[1], hroot, bias1.reshape(1, D))

    parts2 = _sc_scatter(h1_pad, src, dst, edge_norm, scale=False)
    out_pad = _compute_out(parts2[0], parts2[1], h1_pad,
                           w_neigh2, w_root2, bias2.reshape(1, D))
    return out_pad[:N]


# speedup vs baseline: 7.8553x; 7.8553x over previous
import functools

import jax
import jax.numpy as jnp
from jax import lax
from jax.experimental import pallas as pl
from jax.experimental.pallas import tpu as pltpu
from jax.experimental.pallas import tpu_sc as plsc

N = 10000
E = 320000
D = 128
R = 8
NB = 30

NPAD = 10240
NC = 2
NS = 16
NW = NC * NS
EW = E // NW
C = 80
NCHUNK = EW // C
ROWS_PER_TILE = NPAD // NS


def _weights_body(coef_ref, basis_ref, root1_ref, out_ref):
    r = pl.program_id(0)
    acc = jnp.zeros((D, D), jnp.float32)
    for b in range(NB):
        acc = acc + coef_ref[r, b] * basis_ref[b]
    out_ref[0] = jnp.where(r == R, root1_ref[...], acc)


def _compute_w_all(coef_ext, basis, root1):
    return pl.pallas_call(
        _weights_body,
        grid=(R + 1,),
        in_specs=[
            pl.BlockSpec(memory_space=pltpu.SMEM),
            pl.BlockSpec((NB, D, D), lambda r: (0, 0, 0)),
            pl.BlockSpec((D, D), lambda r: (0, 0)),
        ],
        out_specs=pl.BlockSpec((1, D, D), lambda r: (r, 0, 0)),
        out_shape=jax.ShapeDtypeStruct((R + 1, D, D), jnp.float32),
    )(coef_ext, basis, root1)


def _matmul_body(x_ref, w_ref, out_ref):
    out_ref[0] = jnp.dot(x_ref[...], w_ref[0], preferred_element_type=jnp.float32)


def _compute_h_all(x_pad, w_all, bn=512):
    nblk = NPAD // bn
    return pl.pallas_call(
        _matmul_body,
        grid=(R + 1, nblk),
        in_specs=[
            pl.BlockSpec((bn, D), lambda r, n: (n, 0)),
            pl.BlockSpec((1, D, D), lambda r, n: (r, 0, 0)),
        ],
        out_specs=pl.BlockSpec((1, bn, D), lambda r, n: (r, n, 0)),
        out_shape=jax.ShapeDtypeStruct((R + 1, NPAD, D), jnp.float32),
    )(x_pad, w_all)


def _h1_body(p0_ref, p1_ref, hr_ref, bias_ref, out_ref):
    out_ref[...] = p0_ref[...] + p1_ref[...] + hr_ref[...] + bias_ref[...]


def _compute_h1(p0, p1, hroot, bias1, bn=512):
    nblk = NPAD // bn
    blk = pl.BlockSpec((bn, D), lambda n: (n, 0))
    return pl.pallas_call(
        _h1_body,
        grid=(nblk,),
        in_specs=[blk, blk, blk, pl.BlockSpec((1, D), lambda n: (0, 0))],
        out_specs=blk,
        out_shape=jax.ShapeDtypeStruct((NPAD, D), jnp.float32),
    )(p0, p1, hroot, bias1)


def _out_body(q0_ref, q1_ref, h1_ref, wn_ref, wr_ref, bias_ref, out_ref):
    agg = q0_ref[...] + q1_ref[...]
    out_ref[...] = (
        jnp.dot(agg, wn_ref[...], preferred_element_type=jnp.float32)
        + jnp.dot(h1_ref[...], wr_ref[...], preferred_element_type=jnp.float32)
        + bias_ref[...]
    )


def _compute_out(q0, q1, h1, wn, wr, bias2, bn=512):
    nblk = NPAD // bn
    blk = pl.BlockSpec((bn, D), lambda n: (n, 0))
    wblk = pl.BlockSpec((D, D), lambda n: (0, 0))
    return pl.pallas_call(
        _out_body,
        grid=(nblk,),
        in_specs=[blk, blk, blk, wblk, wblk, pl.BlockSpec((1, D), lambda n: (0, 0))],
        out_specs=blk,
        out_shape=jax.ShapeDtypeStruct((NPAD, D), jnp.float32),
    )(q0, q1, h1, wn, wr, bias2)


def _scatter_body(scale, table_hbm, idx_hbm, dst_hbm, norm_hbm, out_hbm,
                  idx_v, dst_v, norm_v, rows_v, zbuf_v, acc_shared, sem):
    c = lax.axis_index("c")
    s = lax.axis_index("s")
    wid = c * NS + s

    zero16 = jnp.zeros((16,), jnp.float32)

    def zrow(i, _):
        for j in range(D // 16):
            zbuf_v[i, pl.ds(j * 16, 16)] = zero16
        return 0

    lax.fori_loop(0, C, zrow, 0)
    for k in range(ROWS_PER_TILE // C):
        pltpu.sync_copy(zbuf_v, acc_shared.at[pl.ds(s * ROWS_PER_TILE + k * C, C), :])
    plsc.subcore_barrier()

    def chunk(g, _):
        base = wid * EW + g * C
        pltpu.sync_copy(idx_hbm.at[pl.ds(base, C)], idx_v)
        pltpu.sync_copy(dst_hbm.at[pl.ds(base, C)], dst_v)
        pltpu.async_copy(table_hbm.at[idx_v], rows_v, sem).wait()
        if scale:
            pltpu.sync_copy(norm_hbm.at[pl.ds(base, C)], norm_v)

            def edge16(t, _):
                nvec = norm_v[pl.ds(t * 16, 16)]
                for l in range(16):
                    nv = nvec[l]
                    row = t * 16 + l
                    for j in range(D // 16):
                        sl = pl.ds(j * 16, 16)
                        rows_v[row, sl] = rows_v[row, sl] * nv
                return 0

            lax.fori_loop(0, C // 16, edge16, 0)
        pltpu.sync_copy(rows_v, acc_shared.at[dst_v], add=True)
        return 0

    lax.fori_loop(0, NCHUNK, chunk, 0)
    plsc.subcore_barrier()

    pltpu.sync_copy(
        acc_shared.at[pl.ds(s * ROWS_PER_TILE, ROWS_PER_TILE), :],
        out_hbm.at[c, pl.ds(s * ROWS_PER_TILE, ROWS_PER_TILE), :],
    )


def _sc_scatter(table, idx, dst, norm, scale):
    mesh = plsc.VectorSubcoreMesh(core_axis_name="c", subcore_axis_name="s")
    kern = pl.kernel(
        functools.partial(_scatter_body, scale),
        out_type=jax.ShapeDtypeStruct((NC, NPAD, D), jnp.float32),
        mesh=mesh,
        scratch_types=[
            pltpu.VMEM((C,), jnp.int32),
            pltpu.VMEM((C,), jnp.int32),
            pltpu.VMEM((C,), jnp.float32),
            pltpu.VMEM((C, D), jnp.float32),
            pltpu.VMEM((C, D), jnp.float32),
            pltpu.VMEM_SHARED((NPAD, D), jnp.float32),
            pltpu.SemaphoreType.DMA,
        ],
    )
    return kern(table, idx, dst, norm)


def kernel(node_features, edge_index, edge_norm, edge_type, basis, coef,
           root1, bias1, w_neigh2, w_root2, bias2):
    src = edge_index[0].astype(jnp.int32)
    dst = edge_index[1].astype(jnp.int32)
    et = edge_type.astype(jnp.int32)
    idx1 = et * NPAD + src

    x_pad = jnp.pad(node_features, ((0, NPAD - N), (0, 0)))
    coef_ext = jnp.concatenate([coef, jnp.zeros((1, NB), coef.dtype)], axis=0)

    w_all = _compute_w_all(coef_ext, basis, root1)
    h_all = _compute_h_all(x_pad, w_all)
    table1 = h_all.reshape((R + 1) * NPAD, D)
    hroot = h_all[R]

    parts1 = _sc_scatter(table1, idx1, dst, edge_norm, scale=True)
    h1_pad = _compute_h1(parts1[0], parts1[1], hroot, bias1.reshape(1, D))

    parts2 = _sc_scatter(h1_pad, src, dst, edge_norm, scale=False)
    out_pad = _compute_out(parts2[0], parts2[1], h1_pad,
                           w_neigh2, w_root2, bias2.reshape(1, D))
    return out_pad[:N]


# speedup vs baseline: 13.6342x; 1.7357x over previous
import functools

import jax
import jax.numpy as jnp
from jax import lax
from jax.experimental import pallas as pl
from jax.experimental.pallas import tpu as pltpu
from jax.experimental.pallas import tpu_sc as plsc

N = 10000
E = 320000
D = 128
R = 8
NB = 30

NPAD = 10240
NC = 2
NS = 16
NW = NC * NS
EW = E // NW
C = 80
NCHUNK = EW // C
ROWS_PER_TILE = NPAD // NS


def _weights_body(coef_ref, basis_ref, root1_ref, out_ref):
    r = pl.program_id(0)
    acc = jnp.zeros((D, D), jnp.float32)
    for b in range(NB):
        acc = acc + coef_ref[r, b] * basis_ref[b]
    out_ref[0] = jnp.where(r == R, root1_ref[...], acc)


def _compute_w_all(coef_ext, basis, root1):
    return pl.pallas_call(
        _weights_body,
        grid=(R + 1,),
        in_specs=[
            pl.BlockSpec(memory_space=pltpu.SMEM),
            pl.BlockSpec((NB, D, D), lambda r: (0, 0, 0)),
            pl.BlockSpec((D, D), lambda r: (0, 0)),
        ],
        out_specs=pl.BlockSpec((1, D, D), lambda r: (r, 0, 0)),
        out_shape=jax.ShapeDtypeStruct((R + 1, D, D), jnp.float32),
    )(coef_ext, basis, root1)


def _matmul_body(x_ref, w_ref, out_ref):
    out_ref[0] = jnp.dot(x_ref[...], w_ref[0], preferred_element_type=jnp.float32)


def _compute_h_all(x_pad, w_all, bn=512):
    nblk = NPAD // bn
    return pl.pallas_call(
        _matmul_body,
        grid=(R + 1, nblk),
        in_specs=[
            pl.BlockSpec((bn, D), lambda r, n: (n, 0)),
            pl.BlockSpec((1, D, D), lambda r, n: (r, 0, 0)),
        ],
        out_specs=pl.BlockSpec((1, bn, D), lambda r, n: (r, n, 0)),
        out_shape=jax.ShapeDtypeStruct((R + 1, NPAD, D), jnp.float32),
    )(x_pad, w_all)


def _h1_body(p0_ref, p1_ref, hr_ref, bias_ref, out_ref):
    out_ref[...] = p0_ref[...] + p1_ref[...] + hr_ref[...] + bias_ref[...]


def _compute_h1(p0, p1, hroot, bias1, bn=512):
    nblk = NPAD // bn
    blk = pl.BlockSpec((bn, D), lambda n: (n, 0))
    return pl.pallas_call(
        _h1_body,
        grid=(nblk,),
        in_specs=[blk, blk, blk, pl.BlockSpec((1, D), lambda n: (0, 0))],
        out_specs=blk,
        out_shape=jax.ShapeDtypeStruct((NPAD, D), jnp.float32),
    )(p0, p1, hroot, bias1)


def _out_body(q0_ref, q1_ref, h1_ref, wn_ref, wr_ref, bias_ref, out_ref):
    agg = q0_ref[...] + q1_ref[...]
    out_ref[...] = (
        jnp.dot(agg, wn_ref[...], preferred_element_type=jnp.float32)
        + jnp.dot(h1_ref[...], wr_ref[...], preferred_element_type=jnp.float32)
        + bias_ref[...]
    )


def _compute_out(q0, q1, h1, wn, wr, bias2, bn=512):
    nblk = NPAD // bn
    blk = pl.BlockSpec((bn, D), lambda n: (n, 0))
    wblk = pl.BlockSpec((D, D), lambda n: (0, 0))
    return pl.pallas_call(
        _out_body,
        grid=(nblk,),
        in_specs=[blk, blk, blk, wblk, wblk, pl.BlockSpec((1, D), lambda n: (0, 0))],
        out_specs=blk,
        out_shape=jax.ShapeDtypeStruct((NPAD, D), jnp.float32),
    )(q0, q1, h1, wn, wr, bias2)


def _scatter_body(scale, table_hbm, idx_hbm, dst_hbm, norm_hbm, out_hbm,
                  idx0, idx1, dst0, dst1, norm0, norm1, rows0, rows1,
                  acc_shared, semm0, semm1, sem0, sem1):
    c = lax.axis_index("c")
    s = lax.axis_index("s")
    wid = c * NS + s

    zero16 = jnp.zeros((16,), jnp.float32)

    def zrow(i, _):
        for j in range(D // 16):
            rows0[i, pl.ds(j * 16, 16)] = zero16
        return 0

    lax.fori_loop(0, C, zrow, 0)
    for k in range(ROWS_PER_TILE // C):
        pltpu.sync_copy(rows0, acc_shared.at[pl.ds(s * ROWS_PER_TILE + k * C, C), :])
    plsc.subcore_barrier()

    def fetch_meta(g, ib, db, nb, sem):
        g = jnp.minimum(g, NCHUNK - 1)
        base = wid * EW + g * C
        pltpu.async_copy(idx_hbm.at[pl.ds(base, C)], ib, sem)
        pltpu.async_copy(dst_hbm.at[pl.ds(base, C)], db, sem)
        if scale:
            pltpu.async_copy(norm_hbm.at[pl.ds(base, C)], nb, sem)

    def wait_meta(g, ib, db, nb, sem):
        g = jnp.minimum(g, NCHUNK - 1)
        base = wid * EW + g * C
        pltpu.make_async_copy(idx_hbm.at[pl.ds(base, C)], ib, sem).wait()
        pltpu.make_async_copy(dst_hbm.at[pl.ds(base, C)], db, sem).wait()
        if scale:
            pltpu.make_async_copy(norm_hbm.at[pl.ds(base, C)], nb, sem).wait()

    def start_gather(ib, rows, sem):
        pltpu.async_copy(table_hbm.at[ib], rows, sem)

    def process(ib, db, nb, rows, sem):
        pltpu.make_async_copy(table_hbm.at[ib], rows, sem).wait()
        if scale:
            def edge16(t, _):
                nvec = nb[pl.ds(t * 16, 16)]
                for l in range(16):
                    nv = nvec[l]
                    row = t * 16 + l
                    for j in range(D // 16):
                        sl = pl.ds(j * 16, 16)
                        rows[row, sl] = rows[row, sl] * nv
                return 0

            lax.fori_loop(0, C // 16, edge16, 0)
        pltpu.sync_copy(rows, acc_shared.at[db], add=True)

    fetch_meta(0, idx0, dst0, norm0, semm0)
    fetch_meta(1, idx1, dst1, norm1, semm1)
    wait_meta(0, idx0, dst0, norm0, semm0)
    start_gather(idx0, rows0, sem0)

    def pair(k, _):
        g0 = 2 * k
        wait_meta(g0 + 1, idx1, dst1, norm1, semm1)
        start_gather(idx1, rows1, sem1)
        process(idx0, dst0, norm0, rows0, sem0)
        fetch_meta(g0 + 2, idx0, dst0, norm0, semm0)
        wait_meta(g0 + 2, idx0, dst0, norm0, semm0)
        start_gather(idx0, rows0, sem0)
        process(idx1, dst1, norm1, rows1, sem1)
        fetch_meta(g0 + 3, idx1, dst1, norm1, semm1)
        return 0

    lax.fori_loop(0, (NCHUNK - 1) // 2, pair, 0)
    process(idx0, dst0, norm0, rows0, sem0)
    wait_meta(NCHUNK - 1, idx1, dst1, norm1, semm1)

    plsc.subcore_barrier()
    pltpu.sync_copy(
        acc_shared.at[pl.ds(s * ROWS_PER_TILE, ROWS_PER_TILE), :],
        out_hbm.at[c, pl.ds(s * ROWS_PER_TILE, ROWS_PER_TILE), :],
    )


def _sc_scatter(table, idx, dst, norm, scale):
    mesh = plsc.VectorSubcoreMesh(core_axis_name="c", subcore_axis_name="s")
    kern = pl.kernel(
        functools.partial(_scatter_body, scale),
        out_type=jax.ShapeDtypeStruct((NC, NPAD, D), jnp.float32),
        mesh=mesh,
        scratch_types=[
            pltpu.VMEM((C,), jnp.int32),
            pltpu.VMEM((C,), jnp.int32),
            pltpu.VMEM((C,), jnp.int32),
            pltpu.VMEM((C,), jnp.int32),
            pltpu.VMEM((C,), jnp.float32),
            pltpu.VMEM((C,), jnp.float32),
            pltpu.VMEM((C, D), jnp.float32),
            pltpu.VMEM((C, D), jnp.float32),
            pltpu.VMEM_SHARED((NPAD, D), jnp.float32),
            pltpu.SemaphoreType.DMA,
            pltpu.SemaphoreType.DMA,
            pltpu.SemaphoreType.DMA,
            pltpu.SemaphoreType.DMA,
        ],
    )
    return kern(table, idx, dst, norm)


def kernel(node_features, edge_index, edge_norm, edge_type, basis, coef,
           root1, bias1, w_neigh2, w_root2, bias2):
    src = edge_index[0].astype(jnp.int32)
    dst = edge_index[1].astype(jnp.int32)
    et = edge_type.astype(jnp.int32)
    idx1 = et * NPAD + src

    x_pad = jnp.pad(node_features, ((0, NPAD - N), (0, 0)))
    coef_ext = jnp.concatenate([coef, jnp.zeros((1, NB), coef.dtype)], axis=0)

    w_all = _compute_w_all(coef_ext, basis, root1)
    h_all = _compute_h_all(x_pad, w_all)
    table1 = h_all.reshape((R + 1) * NPAD, D)
    hroot = h_all[R]

    parts1 = _sc_scatter(table1, idx1, dst, edge_norm, scale=True)
    h1_pad = _compute_h1(parts1[0], parts1[1], hroot, bias1.reshape(1, D))

    parts2 = _sc_scatter(h1_pad, src, dst, edge_norm, scale=False)
    out_pad = _compute_out(parts2[0], parts2[1], h1_pad,
                           w_neigh2, w_root2, bias2.reshape(1, D))
    return out_pad[:N]


# speedup vs baseline: 15.5928x; 1.1437x over previous
import functools

import jax
import jax.numpy as jnp
from jax import lax
from jax.experimental import pallas as pl
from jax.experimental.pallas import tpu as pltpu
from jax.experimental.pallas import tpu_sc as plsc

N = 10000
E = 320000
D = 128
R = 8
NB = 30

NPAD = 10240
NC = 2
NS = 16
NW = NC * NS
EW = E // NW
C = 80
NCHUNK = EW // C
ROWS_PER_TILE = NPAD // NS


def _weights_body(coef_ref, basis_ref, root1_ref, out_ref):
    r = pl.program_id(0)
    acc = jnp.zeros((D, D), jnp.float32)
    for b in range(NB):
        acc = acc + coef_ref[r, b] * basis_ref[b]
    out_ref[0] = jnp.where(r == R, root1_ref[...], acc)


def _compute_w_all(coef_ext, basis, root1):
    return pl.pallas_call(
        _weights_body,
        grid=(R + 1,),
        in_specs=[
            pl.BlockSpec(memory_space=pltpu.SMEM),
            pl.BlockSpec((NB, D, D), lambda r: (0, 0, 0)),
            pl.BlockSpec((D, D), lambda r: (0, 0)),
        ],
        out_specs=pl.BlockSpec((1, D, D), lambda r: (r, 0, 0)),
        out_shape=jax.ShapeDtypeStruct((R + 1, D, D), jnp.float32),
    )(coef_ext, basis, root1)


def _matmul_body(x_ref, w_ref, out_ref):
    out_ref[0] = jnp.dot(x_ref[...], w_ref[0], preferred_element_type=jnp.float32)


def _compute_h_all(x_pad, w_all, bn=512):
    nblk = NPAD // bn
    return pl.pallas_call(
        _matmul_body,
        grid=(R + 1, nblk),
        in_specs=[
            pl.BlockSpec((bn, D), lambda r, n: (n, 0)),
            pl.BlockSpec((1, D, D), lambda r, n: (r, 0, 0)),
        ],
        out_specs=pl.BlockSpec((1, bn, D), lambda r, n: (r, n, 0)),
        out_shape=jax.ShapeDtypeStruct((R + 1, NPAD, D), jnp.float32),
    )(x_pad, w_all)


def _h1_body(p0_ref, p1_ref, hr_ref, bias_ref, out_ref):
    out_ref[...] = p0_ref[...] + p1_ref[...] + hr_ref[...] + bias_ref[...]


def _compute_h1(p0, p1, hroot, bias1, bn=512):
    nblk = NPAD // bn
    blk = pl.BlockSpec((bn, D), lambda n: (n, 0))
    return pl.pallas_call(
        _h1_body,
        grid=(nblk,),
        in_specs=[blk, blk, blk, pl.BlockSpec((1, D), lambda n: (0, 0))],
        out_specs=blk,
        out_shape=jax.ShapeDtypeStruct((NPAD, D), jnp.float32),
    )(p0, p1, hroot, bias1)


def _out_body(q0_ref, q1_ref, h1_ref, wn_ref, wr_ref, bias_ref, out_ref):
    agg = q0_ref[...] + q1_ref[...]
    out_ref[...] = (
        jnp.dot(agg, wn_ref[...], preferred_element_type=jnp.float32)
        + jnp.dot(h1_ref[...], wr_ref[...], preferred_element_type=jnp.float32)
        + bias_ref[...]
    )


def _compute_out(q0, q1, h1, wn, wr, bias2, bn=512):
    nblk = NPAD // bn
    blk = pl.BlockSpec((bn, D), lambda n: (n, 0))
    wblk = pl.BlockSpec((D, D), lambda n: (0, 0))
    return pl.pallas_call(
        _out_body,
        grid=(nblk,),
        in_specs=[blk, blk, blk, wblk, wblk, pl.BlockSpec((1, D), lambda n: (0, 0))],
        out_specs=blk,
        out_shape=jax.ShapeDtypeStruct((NPAD, D), jnp.float32),
    )(q0, q1, h1, wn, wr, bias2)


def _scatter_body(scale, table_hbm, meta_hbm, norm_hbm, out_hbm,
                  meta0, meta1, meta2, normb0, normb1, normb2,
                  idxb0, idxb1, idxb2, dstb0, dstb1, dstb2,
                  rows0, rows1, rows2, acc_shared,
                  semm0, semm1, semm2, semg0, semg1, semg2,
                  sems0, sems1, sems2):
    c = lax.axis_index("c")
    s = lax.axis_index("s")
    wid = c * NS + s

    metas = (meta0, meta1, meta2)
    normbs = (normb0, normb1, normb2)
    idxbs = (idxb0, idxb1, idxb2)
    dstbs = (dstb0, dstb1, dstb2)
    rows = (rows0, rows1, rows2)
    semm = (semm0, semm1, semm2)
    semg = (semg0, semg1, semg2)
    sems = (sems0, sems1, sems2)

    zero16 = jnp.zeros((16,), jnp.float32)

    def zrow(i, _):
        for j in range(D // 16):
            rows0[i, pl.ds(j * 16, 16)] = zero16
        return 0

    lax.fori_loop(0, C, zrow, 0)
    for k in range(ROWS_PER_TILE // C):
        pltpu.sync_copy(rows0, acc_shared.at[pl.ds(s * ROWS_PER_TILE + k * C, C), :])
    plsc.subcore_barrier()

    def meta_src(g):
        g = jnp.minimum(g, NCHUNK - 1)
        return meta_hbm.at[pl.ds((wid * NCHUNK + g) * 2 * C, 2 * C)]

    def norm_src(g):
        g = jnp.minimum(g, NCHUNK - 1)
        return norm_hbm.at[pl.ds(wid * EW + g * C, C)]

    def fetch_meta(g, b):
        pltpu.async_copy(meta_src(g), metas[b], semm[b])
        if scale:
            pltpu.async_copy(norm_src(g), normbs[b], semm[b])

    def wait_meta(g, b):
        pltpu.make_async_copy(meta_src(g), metas[b], semm[b]).wait()
        if scale:
            pltpu.make_async_copy(norm_src(g), normbs[b], semm[b]).wait()

    def gather(g, b):
        wait_meta(g, b)
        for t in range(C // 16):
            sl = pl.ds(t * 16, 16)
            idxbs[b][sl] = metas[b][sl]
        pltpu.async_copy(table_hbm.at[idxbs[b]], rows[b], semg[b])

    def process(b):
        pltpu.make_async_copy(table_hbm.at[idxbs[b]], rows[b], semg[b]).wait()
        if scale:
            def edge16(t, _):
                nvec = normbs[b][pl.ds(t * 16, 16)]
                for l in range(16):
                    nv = nvec[l]
                    row = t * 16 + l
                    for j in range(D // 16):
                        sl = pl.ds(j * 16, 16)
                        rows[b][row, sl] = rows[b][row, sl] * nv
                return 0

            lax.fori_loop(0, C // 16, edge16, 0)
        for t in range(C // 16):
            dstbs[b][pl.ds(t * 16, 16)] = metas[b][pl.ds(C + t * 16, 16)]
        pltpu.async_copy(rows[b], acc_shared.at[dstbs[b]], sems[b], add=True)

    def drain_scatter(b):
        pltpu.make_async_copy(rows[b], acc_shared.at[dstbs[b]], sems[b]).wait()

    fetch_meta(0, 0)
    fetch_meta(1, 1)
    gather(0, 0)
    fetch_meta(2, 2)
    gather(1, 1)

    def triple(k, _):
        g = 3 * k
        process(0)
        fetch_meta(g + 3, 0)

        @pl.when(k > 0)
        def _():
            drain_scatter(2)

        gather(g + 2, 2)
        process(1)
        fetch_meta(g + 4, 1)
        drain_scatter(0)
        gather(g + 3, 0)
        process(2)
        fetch_meta(g + 5, 2)
        drain_scatter(1)
        gather(g + 4, 1)
        return 0

    lax.fori_loop(0, NCHUNK // 3, triple, 0)
    process(0)
    process(1)
    drain_scatter(2)
    drain_scatter(0)
    drain_scatter(1)
    wait_meta(NCHUNK - 1, 2)

    plsc.subcore_barrier()
    pltpu.sync_copy(
        acc_shared.at[pl.ds(s * ROWS_PER_TILE, ROWS_PER_TILE), :],
        out_hbm.at[c, pl.ds(s * ROWS_PER_TILE, ROWS_PER_TILE), :],
    )


def _sc_scatter(table, meta, norm, scale):
    mesh = plsc.VectorSubcoreMesh(core_axis_name="c", subcore_axis_name="s")
    kern = pl.kernel(
        functools.partial(_scatter_body, scale),
        out_type=jax.ShapeDtypeStruct((NC, NPAD, D), jnp.float32),
        mesh=mesh,
        scratch_types=[
            pltpu.VMEM((2 * C,), jnp.int32),
            pltpu.VMEM((2 * C,), jnp.int32),
            pltpu.VMEM((2 * C,), jnp.int32),
            pltpu.VMEM((C,), jnp.float32),
            pltpu.VMEM((C,), jnp.float32),
            pltpu.VMEM((C,), jnp.float32),
            pltpu.VMEM((C,), jnp.int32),
            pltpu.VMEM((C,), jnp.int32),
            pltpu.VMEM((C,), jnp.int32),
            pltpu.VMEM((C,), jnp.int32),
            pltpu.VMEM((C,), jnp.int32),
            pltpu.VMEM((C,), jnp.int32),
            pltpu.VMEM((C, D), jnp.float32),
            pltpu.VMEM((C, D), jnp.float32),
            pltpu.VMEM((C, D), jnp.float32),
            pltpu.VMEM_SHARED((NPAD, D), jnp.float32),
            pltpu.SemaphoreType.DMA,
            pltpu.SemaphoreType.DMA,
            pltpu.SemaphoreType.DMA,
            pltpu.SemaphoreType.DMA,
            pltpu.SemaphoreType.DMA,
            pltpu.SemaphoreType.DMA,
            pltpu.SemaphoreType.DMA,
            pltpu.SemaphoreType.DMA,
            pltpu.SemaphoreType.DMA,
        ],
    )
    return kern(table, meta, norm)


def kernel(node_features, edge_index, edge_norm, edge_type, basis, coef,
           root1, bias1, w_neigh2, w_root2, bias2):
    src = edge_index[0].astype(jnp.int32)
    dst = edge_index[1].astype(jnp.int32)
    et = edge_type.astype(jnp.int32)
    idx1 = et * NPAD + src
    def pack_meta(idx):
        cols = jnp.stack([idx.reshape(-1, C), dst.reshape(-1, C)], axis=1)
        return cols.reshape(-1)

    meta1 = pack_meta(idx1)
    meta2 = pack_meta(src)

    x_pad = jnp.pad(node_features, ((0, NPAD - N), (0, 0)))
    coef_ext = jnp.concatenate([coef, jnp.zeros((1, NB), coef.dtype)], axis=0)

    w_all = _compute_w_all(coef_ext, basis, root1)
    h_all = _compute_h_all(x_pad, w_all)
    table1 = h_all.reshape((R + 1) * NPAD, D)
    hroot = h_all[R]

    parts1 = _sc_scatter(table1, meta1, edge_norm, scale=True)
    h1_pad = _compute_h1(parts1[0], parts1[1], hroot, bias1.reshape(1, D))

    parts2 = _sc_scatter(h1_pad, meta2, edge_norm, scale=False)
    out_pad = _compute_out(parts2[0], parts2[1], h1_pad,
                           w_neigh2, w_root2, bias2.reshape(1, D))
    return out_pad[:N]


# speedup vs baseline: 21.9827x; 1.4098x over previous
import functools

import jax
import jax.numpy as jnp
from jax import lax
from jax.experimental import pallas as pl
from jax.experimental.pallas import tpu as pltpu
from jax.experimental.pallas import tpu_sc as plsc

N = 10000
E = 320000
D = 128
R = 8
NB = 30

NPAD = 10240
NC = 2
NS = 16
NW = NC * NS
EW = E // NW
C = 80
NCHUNK = EW // C
ROWS_PER_TILE = NPAD // NS


def _weights_body(coef_ref, basis_ref, root1_ref, out_ref):
    r = pl.program_id(0)
    acc = jnp.zeros((D, D), jnp.float32)
    for b in range(NB):
        acc = acc + coef_ref[r, b] * basis_ref[b]
    out_ref[0] = jnp.where(r == R, root1_ref[...], acc)


def _compute_w_all(coef_ext, basis, root1):
    return pl.pallas_call(
        _weights_body,
        grid=(R + 1,),
        in_specs=[
            pl.BlockSpec(memory_space=pltpu.SMEM),
            pl.BlockSpec((NB, D, D), lambda r: (0, 0, 0)),
            pl.BlockSpec((D, D), lambda r: (0, 0)),
        ],
        out_specs=pl.BlockSpec((1, D, D), lambda r: (r, 0, 0)),
        out_shape=jax.ShapeDtypeStruct((R + 1, D, D), jnp.float32),
    )(coef_ext, basis, root1)


def _matmul_body(x_ref, w_ref, out_ref):
    for r in range(R + 1):
        out_ref[r] = jnp.dot(x_ref[...], w_ref[r],
                             preferred_element_type=jnp.float32)


def _compute_h_all(x_pad, w_all, bn=1024):
    nblk = NPAD // bn
    return pl.pallas_call(
        _matmul_body,
        grid=(nblk,),
        in_specs=[
            pl.BlockSpec((bn, D), lambda n: (n, 0)),
            pl.BlockSpec((R + 1, D, D), lambda n: (0, 0, 0)),
        ],
        out_specs=pl.BlockSpec((R + 1, bn, D), lambda n: (0, n, 0)),
        out_shape=jax.ShapeDtypeStruct((R + 1, NPAD, D), jnp.float32),
    )(x_pad, w_all)


def _h1_body(p0_ref, p1_ref, hr_ref, bias_ref, out_ref):
    out_ref[...] = p0_ref[...] + p1_ref[...] + hr_ref[...] + bias_ref[...]


def _compute_h1(p0, p1, hroot, bias1, bn=512):
    nblk = NPAD // bn
    blk = pl.BlockSpec((bn, D), lambda n: (n, 0))
    return pl.pallas_call(
        _h1_body,
        grid=(nblk,),
        in_specs=[blk, blk, blk, pl.BlockSpec((1, D), lambda n: (0, 0))],
        out_specs=blk,
        out_shape=jax.ShapeDtypeStruct((NPAD, D), jnp.float32),
    )(p0, p1, hroot, bias1)


def _out_body(q0_ref, q1_ref, h1_ref, wn_ref, wr_ref, bias_ref, out_ref):
    agg = q0_ref[...] + q1_ref[...]
    out_ref[...] = (
        jnp.dot(agg, wn_ref[...], preferred_element_type=jnp.float32)
        + jnp.dot(h1_ref[...], wr_ref[...], preferred_element_type=jnp.float32)
        + bias_ref[...]
    )


def _compute_out(q0, q1, h1, wn, wr, bias2, bn=512):
    nblk = NPAD // bn
    blk = pl.BlockSpec((bn, D), lambda n: (n, 0))
    wblk = pl.BlockSpec((D, D), lambda n: (0, 0))
    return pl.pallas_call(
        _out_body,
        grid=(nblk,),
        in_specs=[blk, blk, blk, wblk, wblk, pl.BlockSpec((1, D), lambda n: (0, 0))],
        out_specs=blk,
        out_shape=jax.ShapeDtypeStruct((NPAD, D), jnp.float32),
    )(q0, q1, h1, wn, wr, bias2)


def _scatter_body(scale, table_hbm, idx_hbm, dst_hbm, norm_hbm, out_hbm,
                  idxf0, idxf1, idxf2, dstf0, dstf1, dstf2,
                  normb0, normb1, normb2, dstb0, dstb1, dstb2,
                  rows0, rows1, rows2, acc_shared,
                  semm0, semm1, semm2, semg0, semg1, semg2,
                  sems0, sems1, sems2):
    c = lax.axis_index("c")
    s = lax.axis_index("s")
    wid = c * NS + s

    idxfs = (idxf0, idxf1, idxf2)
    dstfs = (dstf0, dstf1, dstf2)
    normbs = (normb0, normb1, normb2)
    dstbs = (dstb0, dstb1, dstb2)
    rows = (rows0, rows1, rows2)
    semm = (semm0, semm1, semm2)
    semg = (semg0, semg1, semg2)
    sems = (sems0, sems1, sems2)

    zero16 = jnp.zeros((16,), jnp.float32)

    def zrow(i, _):
        for j in range(D // 16):
            rows0[i, pl.ds(j * 16, 16)] = zero16
        return 0

    lax.fori_loop(0, C, zrow, 0)
    for k in range(ROWS_PER_TILE // C):
        pltpu.sync_copy(rows0, acc_shared.at[pl.ds(s * ROWS_PER_TILE + k * C, C), :])
    plsc.subcore_barrier()

    def chunk_src(arr, g):
        g = jnp.minimum(g, NCHUNK - 1)
        return arr.at[pl.ds(wid * EW + g * C, C)]

    def fetch_meta(g, b):
        pltpu.async_copy(chunk_src(idx_hbm, g), idxfs[b], semm[b])
        pltpu.async_copy(chunk_src(dst_hbm, g), dstfs[b], semm[b])
        if scale:
            pltpu.async_copy(chunk_src(norm_hbm, g), normbs[b], semm[b])

    def wait_meta(g, b):
        pltpu.make_async_copy(chunk_src(idx_hbm, g), idxfs[b], semm[b]).wait()
        pltpu.make_async_copy(chunk_src(dst_hbm, g), dstfs[b], semm[b]).wait()
        if scale:
            pltpu.make_async_copy(chunk_src(norm_hbm, g), normbs[b], semm[b]).wait()

    def gather(g, b):
        wait_meta(g, b)
        pltpu.async_copy(table_hbm.at[idxfs[b]], rows[b], semg[b])

    def process(b):
        pltpu.make_async_copy(table_hbm.at[idxfs[b]], rows[b], semg[b]).wait()
        if scale:
            def edge16(t, _):
                nvec = normbs[b][pl.ds(t * 16, 16)]
                for l in range(16):
                    nv = nvec[l]
                    row = t * 16 + l
                    for j in range(D // 16):
                        sl = pl.ds(j * 16, 16)
                        rows[b][row, sl] = rows[b][row, sl] * nv
                return 0

            lax.fori_loop(0, C // 16, edge16, 0)
        for t in range(C // 16):
            sl = pl.ds(t * 16, 16)
            dstbs[b][sl] = dstfs[b][sl]
        pltpu.async_copy(rows[b], acc_shared.at[dstbs[b]], sems[b], add=True)

    def drain_scatter(b):
        pltpu.make_async_copy(rows[b], acc_shared.at[dstbs[b]], sems[b]).wait()

    fetch_meta(0, 0)
    fetch_meta(1, 1)
    gather(0, 0)
    fetch_meta(2, 2)
    gather(1, 1)

    def triple(k, _):
        g = 3 * k
        process(0)
        fetch_meta(g + 3, 0)

        @pl.when(k > 0)
        def _():
            drain_scatter(2)

        gather(g + 2, 2)
        process(1)
        fetch_meta(g + 4, 1)
        drain_scatter(0)
        gather(g + 3, 0)
        process(2)
        fetch_meta(g + 5, 2)
        drain_scatter(1)
        gather(g + 4, 1)
        return 0

    lax.fori_loop(0, NCHUNK // 3, triple, 0)
    process(0)
    process(1)
    drain_scatter(2)
    drain_scatter(0)
    drain_scatter(1)
    wait_meta(NCHUNK - 1, 2)

    plsc.subcore_barrier()
    pltpu.sync_copy(
        acc_shared.at[pl.ds(s * ROWS_PER_TILE, ROWS_PER_TILE), :],
        out_hbm.at[c, pl.ds(s * ROWS_PER_TILE, ROWS_PER_TILE), :],
    )


def _sc_scatter(table, idx, dst, norm, scale):
    mesh = plsc.VectorSubcoreMesh(core_axis_name="c", subcore_axis_name="s")
    kern = pl.kernel(
        functools.partial(_scatter_body, scale),
        out_type=jax.ShapeDtypeStruct((NC, NPAD, D), jnp.float32),
        mesh=mesh,
        scratch_types=[
            pltpu.VMEM((C,), jnp.int32),
            pltpu.VMEM((C,), jnp.int32),
            pltpu.VMEM((C,), jnp.int32),
            pltpu.VMEM((C,), jnp.int32),
            pltpu.VMEM((C,), jnp.int32),
            pltpu.VMEM((C,), jnp.int32),
            pltpu.VMEM((C,), jnp.float32),
            pltpu.VMEM((C,), jnp.float32),
            pltpu.VMEM((C,), jnp.float32),
            pltpu.VMEM((C,), jnp.int32),
            pltpu.VMEM((C,), jnp.int32),
            pltpu.VMEM((C,), jnp.int32),
            pltpu.VMEM((C, D), jnp.float32),
            pltpu.VMEM((C, D), jnp.float32),
            pltpu.VMEM((C, D), jnp.float32),
            pltpu.VMEM_SHARED((NPAD, D), jnp.float32),
            pltpu.SemaphoreType.DMA,
            pltpu.SemaphoreType.DMA,
            pltpu.SemaphoreType.DMA,
            pltpu.SemaphoreType.DMA,
            pltpu.SemaphoreType.DMA,
            pltpu.SemaphoreType.DMA,
            pltpu.SemaphoreType.DMA,
            pltpu.SemaphoreType.DMA,
            pltpu.SemaphoreType.DMA,
        ],
    )
    return kern(table, idx, dst, norm)


def kernel(node_features, edge_index, edge_norm, edge_type, basis, coef,
           root1, bias1, w_neigh2, w_root2, bias2):
    src = edge_index[0].astype(jnp.int32)
    dst = edge_index[1].astype(jnp.int32)
    et = edge_type.astype(jnp.int32)
    idx1 = et * NPAD + src
    x_pad = jnp.pad(node_features, ((0, NPAD - N), (0, 0)))
    coef_ext = jnp.concatenate([coef, jnp.zeros((1, NB), coef.dtype)], axis=0)

    w_all = _compute_w_all(coef_ext, basis, root1)
    h_all = _compute_h_all(x_pad, w_all)
    table1 = h_all.reshape((R + 1) * NPAD, D)
    hroot = h_all[R]

    parts1 = _sc_scatter(table1, idx1, dst, edge_norm, scale=True)
    h1_pad = _compute_h1(parts1[0], parts1[1], hroot, bias1.reshape(1, D))

    parts2 = _sc_scatter(h1_pad, src, dst, edge_norm, scale=False)
    out_pad = _compute_out(parts2[0], parts2[1], h1_pad,
                           w_neigh2, w_root2, bias2.reshape(1, D))
    return out_pad[:N]


# speedup vs baseline: 22.7806x; 1.0363x over previous
import functools

import jax
import jax.numpy as jnp
from jax import lax
from jax.experimental import pallas as pl
from jax.experimental.pallas import tpu as pltpu
from jax.experimental.pallas import tpu_sc as plsc

N = 10000
E = 320000
D = 128
R = 8
NB = 30

NPAD = 10240
NC = 2
NS = 16
NW = NC * NS
EW = E // NW
C = 80
NCHUNK = EW // C
ROWS_PER_TILE = NPAD // NS


def _weights_body(coef_ref, basis_ref, root1_ref, out_ref):
    r = pl.program_id(0)
    acc = jnp.zeros((D, D), jnp.float32)
    for b in range(NB):
        acc = acc + coef_ref[r, b] * basis_ref[b]
    out_ref[0] = jnp.where(r == R, root1_ref[...], acc)


def _compute_w_all(coef_ext, basis, root1):
    return pl.pallas_call(
        _weights_body,
        grid=(R + 1,),
        in_specs=[
            pl.BlockSpec(memory_space=pltpu.SMEM),
            pl.BlockSpec((NB, D, D), lambda r: (0, 0, 0)),
            pl.BlockSpec((D, D), lambda r: (0, 0)),
        ],
        out_specs=pl.BlockSpec((1, D, D), lambda r: (r, 0, 0)),
        out_shape=jax.ShapeDtypeStruct((R + 1, D, D), jnp.float32),
    )(coef_ext, basis, root1)


def _matmul_body(x_ref, w_ref, out_ref):
    for r in range(R + 1):
        out_ref[r] = jnp.dot(x_ref[...], w_ref[r],
                             preferred_element_type=jnp.float32)


def _compute_h_all(x_pad, w_all, bn=1024):
    nblk = NPAD // bn
    return pl.pallas_call(
        _matmul_body,
        grid=(nblk,),
        in_specs=[
            pl.BlockSpec((bn, D), lambda n: (n, 0)),
            pl.BlockSpec((R + 1, D, D), lambda n: (0, 0, 0)),
        ],
        out_specs=pl.BlockSpec((R + 1, bn, D), lambda n: (0, n, 0)),
        out_shape=jax.ShapeDtypeStruct((R + 1, NPAD, D), jnp.float32),
    )(x_pad, w_all)


def _h1_body(p_ref, hr_ref, bias_ref, out_ref):
    out_ref[...] = p_ref[0] + p_ref[1] + hr_ref[...] + bias_ref[...]


def _compute_h1(parts, hroot, bias1, bn=512):
    nblk = NPAD // bn
    blk = pl.BlockSpec((bn, D), lambda n: (n, 0))
    return pl.pallas_call(
        _h1_body,
        grid=(nblk,),
        in_specs=[pl.BlockSpec((NC, bn, D), lambda n: (0, n, 0)), blk,
                  pl.BlockSpec((1, D), lambda n: (0, 0))],
        out_specs=blk,
        out_shape=jax.ShapeDtypeStruct((NPAD, D), jnp.float32),
    )(parts, hroot, bias1)


def _root_out_body(h1_ref, wr_ref, bias_ref, out_ref):
    out_ref[...] = (
        jnp.dot(h1_ref[...], wr_ref[...], preferred_element_type=jnp.float32)
        + bias_ref[...]
    )


def _compute_root_out(h1, wr, bias2, bn=512):
    nblk = NPAD // bn
    blk = pl.BlockSpec((bn, D), lambda n: (n, 0))
    return pl.pallas_call(
        _root_out_body,
        grid=(nblk,),
        in_specs=[blk, pl.BlockSpec((D, D), lambda n: (0, 0)),
                  pl.BlockSpec((1, D), lambda n: (0, 0))],
        out_specs=blk,
        out_shape=jax.ShapeDtypeStruct((NPAD, D), jnp.float32),
    )(h1, wr, bias2)


def _out_body(q_ref, rpart_ref, wn_ref, out_ref):
    agg = q_ref[0] + q_ref[1]
    out_ref[...] = (
        jnp.dot(agg, wn_ref[...], preferred_element_type=jnp.float32)
        + rpart_ref[...]
    )


def _compute_out(parts, rpart, wn, bn=512):
    nblk = NPAD // bn
    blk = pl.BlockSpec((bn, D), lambda n: (n, 0))
    return pl.pallas_call(
        _out_body,
        grid=(nblk,),
        in_specs=[pl.BlockSpec((NC, bn, D), lambda n: (0, n, 0)), blk,
                  pl.BlockSpec((D, D), lambda n: (0, 0))],
        out_specs=blk,
        out_shape=jax.ShapeDtypeStruct((NPAD, D), jnp.float32),
    )(parts, rpart, wn)


def _scatter_body(scale, dst_off, table_hbm, idx_hbm, dst_hbm, norm_hbm, out_hbm,
                  idxf0, idxf1, idxf2, dstf0, dstf1, dstf2,
                  normb0, normb1, normb2, dstb0, dstb1, dstb2,
                  rows0, rows1, rows2, acc_shared,
                  semm0, semm1, semm2, semg0, semg1, semg2,
                  sems0, sems1, sems2):
    c = lax.axis_index("c")
    s = lax.axis_index("s")
    wid = c * NS + s

    idxfs = (idxf0, idxf1, idxf2)
    dstfs = (dstf0, dstf1, dstf2)
    normbs = (normb0, normb1, normb2)
    dstbs = (dstb0, dstb1, dstb2)
    rows = (rows0, rows1, rows2)
    semm = (semm0, semm1, semm2)
    semg = (semg0, semg1, semg2)
    sems = (sems0, sems1, sems2)

    zero16 = jnp.zeros((16,), jnp.float32)

    def zrow(i, _):
        for j in range(D // 16):
            rows0[i, pl.ds(j * 16, 16)] = zero16
        return 0

    lax.fori_loop(0, C, zrow, 0)
    for k in range(ROWS_PER_TILE // C):
        pltpu.sync_copy(rows0, acc_shared.at[pl.ds(s * ROWS_PER_TILE + k * C, C), :])
    plsc.subcore_barrier()

    def chunk_src(arr, g, off=0):
        g = jnp.minimum(g, NCHUNK - 1)
        return arr.at[pl.ds(off + wid * EW + g * C, C)]

    def fetch_meta(g, b):
        pltpu.async_copy(chunk_src(idx_hbm, g), idxfs[b], semm[b])
        pltpu.async_copy(chunk_src(dst_hbm, g, dst_off), dstfs[b], semm[b])
        if scale:
            pltpu.async_copy(chunk_src(norm_hbm, g), normbs[b], semm[b])

    def wait_meta(g, b):
        pltpu.make_async_copy(chunk_src(idx_hbm, g), idxfs[b], semm[b]).wait()
        pltpu.make_async_copy(chunk_src(dst_hbm, g, dst_off), dstfs[b], semm[b]).wait()
        if scale:
            pltpu.make_async_copy(chunk_src(norm_hbm, g), normbs[b], semm[b]).wait()

    def gather(g, b):
        wait_meta(g, b)
        pltpu.async_copy(table_hbm.at[idxfs[b]], rows[b], semg[b])

    def process(b):
        pltpu.make_async_copy(table_hbm.at[idxfs[b]], rows[b], semg[b]).wait()
        if scale:
            def edge16(t, _):
                nvec = normbs[b][pl.ds(t * 16, 16)]
                for l in range(16):
                    nv = nvec[l]
                    row = t * 16 + l
                    for j in range(D // 16):
                        sl = pl.ds(j * 16, 16)
                        rows[b][row, sl] = rows[b][row, sl] * nv
                return 0

            lax.fori_loop(0, C // 16, edge16, 0)
        for t in range(C // 16):
            sl = pl.ds(t * 16, 16)
            dstbs[b][sl] = dstfs[b][sl]
        pltpu.async_copy(rows[b], acc_shared.at[dstbs[b]], sems[b], add=True)

    def drain_scatter(b):
        pltpu.make_async_copy(rows[b], acc_shared.at[dstbs[b]], sems[b]).wait()

    fetch_meta(0, 0)
    fetch_meta(1, 1)
    gather(0, 0)
    fetch_meta(2, 2)
    gather(1, 1)

    def triple(k, _):
        g = 3 * k
        process(0)
        fetch_meta(g + 3, 0)

        @pl.when(k > 0)
        def _():
            drain_scatter(2)

        gather(g + 2, 2)
        process(1)
        fetch_meta(g + 4, 1)
        drain_scatter(0)
        gather(g + 3, 0)
        process(2)
        fetch_meta(g + 5, 2)
        drain_scatter(1)
        gather(g + 4, 1)
        return 0

    lax.fori_loop(0, NCHUNK // 3, triple, 0)
    process(0)
    process(1)
    drain_scatter(2)
    drain_scatter(0)
    drain_scatter(1)
    wait_meta(NCHUNK - 1, 2)

    plsc.subcore_barrier()
    pltpu.sync_copy(
        acc_shared.at[pl.ds(s * ROWS_PER_TILE, ROWS_PER_TILE), :],
        out_hbm.at[c, pl.ds(s * ROWS_PER_TILE, ROWS_PER_TILE), :],
    )


def _sc_scatter(table, idx, dst, norm, scale, dst_off=0):
    mesh = plsc.VectorSubcoreMesh(core_axis_name="c", subcore_axis_name="s")
    kern = pl.kernel(
        functools.partial(_scatter_body, scale, dst_off),
        out_type=jax.ShapeDtypeStruct((NC, NPAD, D), jnp.float32),
        mesh=mesh,
        scratch_types=[
            pltpu.VMEM((C,), jnp.int32),
            pltpu.VMEM((C,), jnp.int32),
            pltpu.VMEM((C,), jnp.int32),
            pltpu.VMEM((C,), jnp.int32),
            pltpu.VMEM((C,), jnp.int32),
            pltpu.VMEM((C,), jnp.int32),
            pltpu.VMEM((C,), jnp.float32),
            pltpu.VMEM((C,), jnp.float32),
            pltpu.VMEM((C,), jnp.float32),
            pltpu.VMEM((C,), jnp.int32),
            pltpu.VMEM((C,), jnp.int32),
            pltpu.VMEM((C,), jnp.int32),
            pltpu.VMEM((C, D), jnp.float32),
            pltpu.VMEM((C, D), jnp.float32),
            pltpu.VMEM((C, D), jnp.float32),
            pltpu.VMEM_SHARED((NPAD, D), jnp.float32),
            pltpu.SemaphoreType.DMA,
            pltpu.SemaphoreType.DMA,
            pltpu.SemaphoreType.DMA,
            pltpu.SemaphoreType.DMA,
            pltpu.SemaphoreType.DMA,
            pltpu.SemaphoreType.DMA,
            pltpu.SemaphoreType.DMA,
            pltpu.SemaphoreType.DMA,
            pltpu.SemaphoreType.DMA,
        ],
    )
    return kern(table, idx, dst, norm)


def kernel(node_features, edge_index, edge_norm, edge_type, basis, coef,
           root1, bias1, w_neigh2, w_root2, bias2):
    edge_flat = edge_index.astype(jnp.int32).reshape(2 * E)
    idx1 = edge_type.astype(jnp.int32) * NPAD + edge_flat[:E]
    x_pad = jnp.pad(node_features, ((0, NPAD - N), (0, 0)))
    coef_ext = jnp.concatenate([coef, jnp.zeros((1, NB), coef.dtype)], axis=0)

    w_all = _compute_w_all(coef_ext, basis, root1)
    h_all = _compute_h_all(x_pad, w_all)
    table1 = h_all.reshape((R + 1) * NPAD, D)
    hroot = h_all[R]

    parts1 = _sc_scatter(table1, idx1, edge_flat, edge_norm, scale=True,
                         dst_off=E)
    h1_pad = _compute_h1(parts1, hroot, bias1.reshape(1, D))

    parts2 = _sc_scatter(h1_pad, edge_flat, edge_flat, edge_norm,
                         scale=False, dst_off=E)
    rpart = _compute_root_out(h1_pad, w_root2, bias2.reshape(1, D))
    out_pad = _compute_out(parts2, rpart, w_neigh2)
    return out_pad[:N]
